# Initial kernel scaffold; baseline (speedup 1.0000x reference)
import jax
import jax.numpy as jnp
from jax.experimental import pallas as pl


def kernel(x, num_people, table):
    raise NotImplementedError("write your pallas kernel here")


# speedup vs baseline: 1.6157x; 1.6157x over previous
import jax
import jax.numpy as jnp
from jax.experimental import pallas as pl
from jax.experimental.pallas import tpu as pltpu

_SEQ_LEN = 200


def _add_emb_kernel(idx_ref, x_ref, t_ref, o_ref):
    row = t_ref[0, 0, :]
    norm = jnp.sqrt(jnp.sum(row * row))
    scale = jnp.where(norm > 1.0, 1.0 / (norm + 1e-7), 1.0)
    o_ref[...] = x_ref[...] + row * scale


def kernel(x, num_people, table):
    total, b, d = x.shape
    n_blocks = total // _SEQ_LEN
    idx = jnp.minimum(jnp.arange(n_blocks, dtype=jnp.int32),
                      jnp.asarray(num_people, jnp.int32) - 1)
    table3 = table.reshape(table.shape[0], 1, d)
    grid_spec = pltpu.PrefetchScalarGridSpec(
        num_scalar_prefetch=1,
        grid=(n_blocks,),
        in_specs=[
            pl.BlockSpec((_SEQ_LEN, b, d), lambda p, idx_ref: (p, 0, 0)),
            pl.BlockSpec((1, 1, d), lambda p, idx_ref: (idx_ref[p], 0, 0)),
        ],
        out_specs=pl.BlockSpec((_SEQ_LEN, b, d), lambda p, idx_ref: (p, 0, 0)),
    )
    return pl.pallas_call(
        _add_emb_kernel,
        grid_spec=grid_spec,
        out_shape=jax.ShapeDtypeStruct(x.shape, x.dtype),
    )(idx, x, table3)


# speedup vs baseline: 1.7135x; 1.0605x over previous
import jax
import jax.numpy as jnp
from jax.experimental import pallas as pl
from jax.experimental.pallas import tpu as pltpu

_SEQ_LEN = 200


def _add_emb_kernel(idx_ref, x_ref, t_ref, o_ref):
    i = idx_ref[pl.program_id(0)]
    row = t_ref[pl.ds(i, 1), :]
    norm = jnp.sqrt(jnp.sum(row * row))
    scale = jnp.where(norm > 1.0, 1.0 / (norm + 1e-7), 1.0)
    o_ref[...] = x_ref[...] + row * scale


def kernel(x, num_people, table):
    total, b, d = x.shape
    n_blocks = total // _SEQ_LEN
    idx = jnp.minimum(jnp.arange(n_blocks, dtype=jnp.int32),
                      jnp.asarray(num_people, jnp.int32) - 1)
    grid_spec = pltpu.PrefetchScalarGridSpec(
        num_scalar_prefetch=1,
        grid=(n_blocks,),
        in_specs=[
            pl.BlockSpec((_SEQ_LEN, b, d), lambda p, idx_ref: (p, 0, 0)),
            pl.BlockSpec(table.shape, lambda p, idx_ref: (0, 0)),
        ],
        out_specs=pl.BlockSpec((_SEQ_LEN, b, d), lambda p, idx_ref: (p, 0, 0)),
    )
    return pl.pallas_call(
        _add_emb_kernel,
        grid_spec=grid_spec,
        out_shape=jax.ShapeDtypeStruct(x.shape, x.dtype),
    )(idx, x, table)
